# Initial kernel scaffold; baseline (speedup 1.0000x reference)
import jax
import jax.numpy as jnp
from jax.experimental import pallas as pl


def kernel(values, sequence_ids, W, b):
    raise NotImplementedError("write your pallas kernel here")


# speedup vs baseline: 1.0069x; 1.0069x over previous
import jax
import jax.numpy as jnp
from jax.experimental import pallas as pl


def _linear_kernel(x_ref, w_ref, b_ref, o_ref):
    o_ref[...] = jax.lax.dot_general(
        x_ref[...], w_ref[...],
        dimension_numbers=(((1,), (1,)), ((), ())),
        preferred_element_type=jnp.float32,
    ) + b_ref[...]


def kernel(values, sequence_ids, W, b):
    del sequence_ids
    B, S, K = values.shape
    N = W.shape[0]
    M = B * S
    x = values.reshape(M, K)
    BM = 1024
    out = pl.pallas_call(
        _linear_kernel,
        grid=(M // BM,),
        in_specs=[
            pl.BlockSpec((BM, K), lambda i: (i, 0)),
            pl.BlockSpec((N, K), lambda i: (0, 0)),
            pl.BlockSpec((1, N), lambda i: (0, 0)),
        ],
        out_specs=pl.BlockSpec((BM, N), lambda i: (i, 0)),
        out_shape=jax.ShapeDtypeStruct((M, N), jnp.float32),
    )(x, W, b.reshape(1, N))
    return out.reshape(B, S, N)


# speedup vs baseline: 1.0088x; 1.0019x over previous
import jax
import jax.numpy as jnp
from jax.experimental import pallas as pl


def _linear_kernel(x_ref, w_ref, b_ref, o_ref):
    o_ref[...] = jax.lax.dot_general(
        x_ref[...].astype(jnp.bfloat16), w_ref[...].astype(jnp.bfloat16),
        dimension_numbers=(((1,), (1,)), ((), ())),
        preferred_element_type=jnp.float32,
    ) + b_ref[...]


def kernel(values, sequence_ids, W, b):
    del sequence_ids
    B, S, K = values.shape
    N = W.shape[0]
    M = B * S
    x = values.reshape(M, K)
    BM = 1024
    out = pl.pallas_call(
        _linear_kernel,
        grid=(M // BM,),
        in_specs=[
            pl.BlockSpec((BM, K), lambda i: (i, 0)),
            pl.BlockSpec((N, K), lambda i: (0, 0)),
            pl.BlockSpec((1, N), lambda i: (0, 0)),
        ],
        out_specs=pl.BlockSpec((BM, N), lambda i: (i, 0)),
        out_shape=jax.ShapeDtypeStruct((M, N), jnp.float32),
    )(x, W, b.reshape(1, N))
    return out.reshape(B, S, N)


# speedup vs baseline: 1.1051x; 1.0954x over previous
import jax
import jax.numpy as jnp
from jax.experimental import pallas as pl


def _linear_kernel(x_ref, w_ref, b_ref, o_ref):
    o_ref[...] = jax.lax.dot_general(
        x_ref[...].astype(jnp.bfloat16), w_ref[...].astype(jnp.bfloat16),
        dimension_numbers=(((1,), (1,)), ((), ())),
        preferred_element_type=jnp.float32,
    ) + b_ref[...]


def kernel(values, sequence_ids, W, b):
    del sequence_ids
    B, S, K = values.shape
    N = W.shape[0]
    M = B * S
    x = values.reshape(M, K)
    BM = 2048
    out = pl.pallas_call(
        _linear_kernel,
        grid=(M // BM,),
        in_specs=[
            pl.BlockSpec((BM, K), lambda i: (i, 0)),
            pl.BlockSpec((N, K), lambda i: (0, 0)),
            pl.BlockSpec((1, N), lambda i: (0, 0)),
        ],
        out_specs=pl.BlockSpec((BM, N), lambda i: (i, 0)),
        out_shape=jax.ShapeDtypeStruct((M, N), jnp.float32),
    )(x, W, b.reshape(1, N))
    return out.reshape(B, S, N)
